# baseline (speedup 1.0000x reference)
import functools

import jax
import jax.numpy as jnp
from jax import lax
from jax.experimental import pallas as pl
from jax.experimental.pallas import tpu as pltpu
from jax.experimental.pallas import tpu_sc as plsc

N = 10000
E = 320000
H = 128
G = 64
_BN_C = 0.9999950000374997

NC, NS, L = 2, 16, 16
NW = NC * NS
NPAD = 10240
EPW = E // NW
CH = 80
NCHUNK = EPW // CH
ROWS_PT = NPAD // NS
DCH = 80
DNCH = EPW // DCH

_MESH = plsc.VectorSubcoreMesh(core_axis_name="c", subcore_axis_name="s",
                               num_cores=NC, num_subcores=NS)


@functools.partial(
    pl.kernel,
    out_type=jax.ShapeDtypeStruct((NW, NPAD), jnp.float32),
    mesh=_MESH,
    compiler_params=pltpu.CompilerParams(needs_layout_passes=False),
    scratch_types=[
        pltpu.VMEM((NPAD,), jnp.float32),
        pltpu.VMEM((DNCH, DCH), jnp.int32),
    ],
)
def _deg_kernel(dst3_hbm, out_hbm, hist_v, dstall):
    c = lax.axis_index("c")
    s = lax.axis_index("s")
    wid = c * NS + s

    pltpu.sync_copy(dst3_hbm.at[wid], dstall)

    def zero_body(i, carry):
        hist_v[pl.ds(i * L, L)] = jnp.zeros((L,), jnp.float32)
        return carry
    lax.fori_loop(0, NPAD // L, zero_body, 0)

    ones = jnp.ones((L,), jnp.float32)

    def chunk_body(i, carry):
        for j in range(DCH // L):
            idx = dstall[i, pl.ds(j * L, L)]
            plsc.addupdate_scatter(hist_v, [idx], ones)
        return carry
    lax.fori_loop(0, DNCH, chunk_body, 0)

    pltpu.sync_copy(hist_v, out_hbm.at[wid])


_NBUF = 4
_NIDX = 8


@functools.partial(
    pl.kernel,
    out_type=jax.ShapeDtypeStruct((NC, NPAD, H), jnp.float32),
    mesh=_MESH,
    scratch_types=[
        [pltpu.VMEM((CH,), jnp.int32)] * _NIDX,
        [pltpu.VMEM((CH,), jnp.int32)] * _NIDX,
        [pltpu.VMEM((CH, H), jnp.float32)] * _NBUF,
        [pltpu.SemaphoreType.DMA] * _NIDX,
        [pltpu.SemaphoreType.DMA] * _NBUF,
        [pltpu.SemaphoreType.DMA] * _NBUF,
        pltpu.VMEM_SHARED((NPAD, H), jnp.float32),
    ],
)
def _agg_kernel(hh_hbm, src_hbm, dst_hbm, out_hbm, srcv, dstv, rows,
                isem, gsem, ssem, acc_sh):
    c = lax.axis_index("c")
    s = lax.axis_index("s")
    wid = c * NS + s
    ebase = wid * EPW

    def zrow(r, carry):
        for j in range(H // L):
            rows[0][r, pl.ds(j * L, L)] = jnp.zeros((L,), jnp.float32)
        return carry
    lax.fori_loop(0, CH, zrow, 0)
    for k in range(ROWS_PT // CH):
        pltpu.sync_copy(rows[0], acc_sh.at[pl.ds(s * ROWS_PT + k * CH, CH)])
    plsc.subcore_barrier()

    def start_idx(i, bi):
        pltpu.async_copy(src_hbm.at[pl.ds(ebase + i * CH, CH)], srcv[bi],
                         isem[bi])
        pltpu.async_copy(dst_hbm.at[pl.ds(ebase + i * CH, CH)], dstv[bi],
                         isem[bi])

    def wait_idx(i, bi):
        pltpu.make_async_copy(src_hbm.at[pl.ds(ebase + i * CH, CH)],
                              srcv[bi], isem[bi]).wait()
        pltpu.make_async_copy(dst_hbm.at[pl.ds(ebase + i * CH, CH)],
                              dstv[bi], isem[bi]).wait()

    def start_gather(bi, br):
        pltpu.async_copy(hh_hbm.at[srcv[bi]], rows[br], gsem[br])

    def wait_gather(bi, br):
        pltpu.make_async_copy(hh_hbm.at[srcv[bi]], rows[br],
                              gsem[br]).wait()

    def start_scatter(bi, br):
        pltpu.async_copy(rows[br], acc_sh.at[dstv[bi]], ssem[br], add=True)

    def wait_scatter(bi, br):
        pltpu.make_async_copy(rows[br], acc_sh.at[dstv[bi]],
                              ssem[br]).wait()

    for j in range(6):
        start_idx(j, j)
    for j in range(2):
        wait_idx(j, j)
        start_gather(j, j)
    for i in range(2):
        wait_idx(i + 2, i + 2)
        start_gather(i + 2, (i + 2) % _NBUF)
        start_idx(i + 6, i + 6)
        wait_gather(i, i)
        start_scatter(i, i)

    n_main = 112
    assert n_main % (_NIDX) == 0

    def group(g, carry):
        for k in range(_NIDX):
            i = 2 + g * _NIDX + k
            br2 = (4 + k) % _NBUF
            bi2 = (4 + k) % _NIDX
            bi6 = k
            br0 = (2 + k) % _NBUF
            bi0 = (2 + k) % _NIDX
            wait_scatter(k, br2)
            wait_idx(i + 2, bi2)
            start_gather(bi2, br2)
            start_idx(i + 6, bi6)
            wait_gather(bi0, br0)
            start_scatter(bi0, br0)
        return carry
    lax.fori_loop(0, n_main // _NIDX, group, 0)

    for i in range(114, NCHUNK):
        br2 = (i + 2) % _NBUF
        bi2 = (i + 2) % _NIDX
        br0 = i % _NBUF
        bi0 = i % _NIDX
        if i + 2 < NCHUNK:
            wait_scatter((i - 2) % _NIDX, (i - 2) % _NBUF)
            wait_idx(i + 2, bi2)
            start_gather(bi2, br2)
        if i + 6 < NCHUNK:
            start_idx(i + 6, (i + 6) % _NIDX)
        wait_gather(bi0, br0)
        start_scatter(bi0, br0)
    for i in range(NCHUNK - 4, NCHUNK):
        wait_scatter(i % _NIDX, i % _NBUF)

    plsc.subcore_barrier()
    pltpu.sync_copy(acc_sh.at[pl.ds(s * ROWS_PT, ROWS_PT)],
                    out_hbm.at[c, pl.ds(s * ROWS_PT, ROWS_PT)])


_BLK = 1000


def _dinv_body(hist_ref, o_ref):
    deg = 1.0 + jnp.sum(hist_ref[...], axis=0, keepdims=True)
    o_ref[...] = lax.rsqrt(deg)


def _dinv_from_hists(hists):
    return pl.pallas_call(
        _dinv_body,
        in_specs=[pl.BlockSpec((NW, NPAD), lambda: (0, 0))],
        out_specs=pl.BlockSpec((1, NPAD), lambda: (0, 0)),
        out_shape=jax.ShapeDtypeStruct((1, NPAD), jnp.float32),
    )(hists)


def _mm_scale_body(x_ref, w_ref, dinv_ref, o_ref):
    o_ref[...] = jnp.dot(x_ref[...], w_ref[...],
                         preferred_element_type=jnp.float32) * dinv_ref[...]


def _mm_scale(x, w, dinv2):
    return pl.pallas_call(
        _mm_scale_body,
        grid=(N // _BLK,),
        in_specs=[
            pl.BlockSpec((_BLK, H), lambda i: (i, 0)),
            pl.BlockSpec((H, H), lambda i: (0, 0)),
            pl.BlockSpec((_BLK, 1), lambda i: (i, 0)),
        ],
        out_specs=pl.BlockSpec((_BLK, H), lambda i: (i, 0)),
        out_shape=jax.ShapeDtypeStruct((N, H), jnp.float32),
    )(x, w, dinv2)


def _mid_body(acc_ref, hh_ref, dinv_ref, w_ref, b_ref, g_ref, be_ref, o_ref):
    dinv = dinv_ref[...]
    a = (acc_ref[0] + acc_ref[1] + hh_ref[...]) * dinv + b_ref[...]
    z = jnp.maximum(a * _BN_C * g_ref[...] + be_ref[...], 0.0)
    o_ref[...] = jnp.dot(z, w_ref[...],
                         preferred_element_type=jnp.float32) * dinv


def _mid_layer(accs, hh, dinv2, w_next, b, g, be):
    return pl.pallas_call(
        _mid_body,
        grid=(N // _BLK,),
        in_specs=[
            pl.BlockSpec((NC, _BLK, H), lambda i: (0, i, 0)),
            pl.BlockSpec((_BLK, H), lambda i: (i, 0)),
            pl.BlockSpec((_BLK, 1), lambda i: (i, 0)),
            pl.BlockSpec((H, H), lambda i: (0, 0)),
            pl.BlockSpec((H,), lambda i: (0,)),
            pl.BlockSpec((H,), lambda i: (0,)),
            pl.BlockSpec((H,), lambda i: (0,)),
        ],
        out_specs=pl.BlockSpec((_BLK, H), lambda i: (i, 0)),
        out_shape=jax.ShapeDtypeStruct((N, H), jnp.float32),
    )(accs, hh, dinv2, w_next, b, g, be)


def _head_body(acc_ref, hh_ref, dinv_ref, b_ref, g_ref, be_ref,
               brow_ref, bcol_ref, wm1_ref, bm1_ref, wm2_ref, bm2_ref,
               wm3_ref, bm3_ref, o_ref, mx_ref, sm_ref):
    a = (acc_ref[0] + acc_ref[1] + hh_ref[...]) * dinv_ref[...] + b_ref[...]
    h = jnp.maximum(a * _BN_C * g_ref[...] + be_ref[...], 0.0)
    brow = brow_ref[...]
    bcol = bcol_ref[...]
    gids = lax.broadcasted_iota(jnp.int32, (G, N), 0)
    onehot = (brow == gids).astype(jnp.float32)
    cnt = jnp.sum(onehot, axis=1, keepdims=True)

    def body(g, carry):
        mask = bcol == g
        m = jnp.where(mask, h, -jnp.inf)
        mx_ref[pl.ds(g, 1), :] = jnp.max(m, axis=0, keepdims=True)
        sm_ref[pl.ds(g, 1), :] = jnp.sum(jnp.where(mask, h, 0.0), axis=0,
                                         keepdims=True)
        return carry
    lax.fori_loop(0, G, body, 0)

    mean = sm_ref[...] / jnp.maximum(cnt, 1.0)
    xg = jnp.concatenate([mean, mx_ref[...]], axis=1)
    o = jnp.maximum(jnp.dot(xg, wm1_ref[...],
                            preferred_element_type=jnp.float32)
                    + bm1_ref[...], 0.0)
    o = jnp.maximum(jnp.dot(o, wm2_ref[...],
                            preferred_element_type=jnp.float32)
                    + bm2_ref[...], 0.0)
    o = jnp.dot(o, wm3_ref[...], preferred_element_type=jnp.float32) \
        + bm3_ref[...]
    o_ref[...] = o[:, 0]


def _head(accs, hh, dinv2, b3, g3, be3, batch, Wm1, bm1, Wm2, bm2, Wm3, bm3):
    return pl.pallas_call(
        _head_body,
        grid=(1,),
        in_specs=[
            pl.BlockSpec((NC, N, H), lambda i: (0, 0, 0)),
            pl.BlockSpec((N, H), lambda i: (0, 0)),
            pl.BlockSpec((N, 1), lambda i: (0, 0)),
            pl.BlockSpec((H,), lambda i: (0,)),
            pl.BlockSpec((H,), lambda i: (0,)),
            pl.BlockSpec((H,), lambda i: (0,)),
            pl.BlockSpec((1, N), lambda i: (0, 0)),
            pl.BlockSpec((N, 1), lambda i: (0, 0)),
            pl.BlockSpec((2 * H, 256), lambda i: (0, 0)),
            pl.BlockSpec((256,), lambda i: (0,)),
            pl.BlockSpec((256, 64), lambda i: (0, 0)),
            pl.BlockSpec((64,), lambda i: (0,)),
            pl.BlockSpec((64, 1), lambda i: (0, 0)),
            pl.BlockSpec((1,), lambda i: (0,)),
        ],
        out_specs=pl.BlockSpec((G,), lambda i: (0,)),
        out_shape=jax.ShapeDtypeStruct((G,), jnp.float32),
        scratch_shapes=[pltpu.VMEM((G, H), jnp.float32),
                        pltpu.VMEM((G, H), jnp.float32)],
    )(accs, hh, dinv2, b3, g3, be3, batch[None, :], batch[:, None],
      Wm1, bm1, Wm2, bm2, Wm3, bm3)


def kernel(x, edge_index, batch, W1, b1, g1, be1, W2, b2, g2, be2,
           W3, b3, g3, be3, Wm1, bm1, Wm2, bm2, Wm3, bm3):
    src = edge_index[0]
    dst = edge_index[1]
    dst3d = dst.reshape(NW, DNCH, DCH)

    hists = _deg_kernel(dst3d)
    dinv_row = _dinv_from_hists(hists)
    dinv2 = dinv_row[0, :N][:, None]

    hh = _mm_scale(x, W1, dinv2)
    accs1 = _agg_kernel(hh, src, dst)
    hh2 = _mid_layer(accs1, hh, dinv2, W2, b1, g1, be1)
    accs2 = _agg_kernel(hh2, src, dst)
    hh3 = _mid_layer(accs2, hh2, dinv2, W3, b2, g2, be2)
    accs3 = _agg_kernel(hh3, src, dst)
    return _head(accs3, hh3, dinv2, b3, g3, be3, batch,
                 Wm1, bm1, Wm2, bm2, Wm3, bm3)


# speedup vs baseline: 1.0561x; 1.0561x over previous
import functools

import jax
import jax.numpy as jnp
from jax import lax
from jax.experimental import pallas as pl
from jax.experimental.pallas import tpu as pltpu
from jax.experimental.pallas import tpu_sc as plsc

N = 10000
E = 320000
H = 128
G = 64
_BN_C = 0.9999950000374997

NC, NS, L = 2, 16, 16
NW = NC * NS
NPAD = 10240
EPW = E // NW
CH = 80
NCHUNK = EPW // CH
ROWS_PT = NPAD // NS
DCH = 80
DNCH = EPW // DCH

_MESH = plsc.VectorSubcoreMesh(core_axis_name="c", subcore_axis_name="s",
                               num_cores=NC, num_subcores=NS)


@functools.partial(
    pl.kernel,
    out_type=jax.ShapeDtypeStruct((NW, NPAD), jnp.float32),
    mesh=_MESH,
    compiler_params=pltpu.CompilerParams(needs_layout_passes=False),
    scratch_types=[
        pltpu.VMEM((NPAD,), jnp.float32),
        pltpu.VMEM((DNCH, DCH), jnp.int32),
    ],
)
def _deg_kernel(dst3_hbm, out_hbm, hist_v, dstall):
    c = lax.axis_index("c")
    s = lax.axis_index("s")
    wid = c * NS + s

    pltpu.sync_copy(dst3_hbm.at[wid], dstall)

    def zero_body(i, carry):
        hist_v[pl.ds(i * L, L)] = jnp.zeros((L,), jnp.float32)
        return carry
    lax.fori_loop(0, NPAD // L, zero_body, 0)

    ones = jnp.ones((L,), jnp.float32)

    def chunk_body(i, carry):
        for j in range(DCH // L):
            idx = dstall[i, pl.ds(j * L, L)]
            plsc.addupdate_scatter(hist_v, [idx], ones)
        return carry
    lax.fori_loop(0, DNCH, chunk_body, 0)

    pltpu.sync_copy(hist_v, out_hbm.at[wid])


_NBUF = 4


@functools.partial(
    pl.kernel,
    out_type=jax.ShapeDtypeStruct((NC, NPAD, H), jnp.float32),
    mesh=_MESH,
    scratch_types=[
        [pltpu.VMEM((CH,), jnp.int32)] * _NBUF,
        [pltpu.VMEM((CH,), jnp.int32)] * _NBUF,
        [pltpu.VMEM((CH, H), jnp.float32)] * _NBUF,
        [pltpu.SemaphoreType.DMA] * _NBUF,
        [pltpu.SemaphoreType.DMA] * _NBUF,
        [pltpu.SemaphoreType.DMA] * _NBUF,
        pltpu.VMEM_SHARED((NPAD, H), jnp.float32),
    ],
)
def _agg_kernel(hh_hbm, src_hbm, dst_hbm, out_hbm, srcv, dstv, rows,
                isem, gsem, ssem, acc_sh):
    c = lax.axis_index("c")
    s = lax.axis_index("s")
    wid = c * NS + s
    ebase = wid * EPW

    def zrow(r, carry):
        for j in range(H // L):
            rows[0][r, pl.ds(j * L, L)] = jnp.zeros((L,), jnp.float32)
        return carry
    lax.fori_loop(0, CH, zrow, 0)
    for k in range(ROWS_PT // CH):
        pltpu.sync_copy(rows[0], acc_sh.at[pl.ds(s * ROWS_PT + k * CH, CH)])
    plsc.subcore_barrier()

    def start_idx(i, b):
        pltpu.async_copy(src_hbm.at[pl.ds(ebase + i * CH, CH)], srcv[b],
                         isem[b])
        pltpu.async_copy(dst_hbm.at[pl.ds(ebase + i * CH, CH)], dstv[b],
                         isem[b])

    def wait_idx(i, b):
        pltpu.make_async_copy(src_hbm.at[pl.ds(ebase + i * CH, CH)],
                              srcv[b], isem[b]).wait()
        pltpu.make_async_copy(dst_hbm.at[pl.ds(ebase + i * CH, CH)],
                              dstv[b], isem[b]).wait()

    def start_gather(i, b):
        pltpu.async_copy(hh_hbm.at[srcv[b]], rows[b], gsem[b])

    def wait_gather(i, b):
        pltpu.make_async_copy(hh_hbm.at[srcv[b]], rows[b], gsem[b]).wait()

    def start_scatter(i, b):
        pltpu.async_copy(rows[b], acc_sh.at[dstv[b]], ssem[b], add=True)

    def wait_scatter(i, b):
        pltpu.make_async_copy(rows[b], acc_sh.at[dstv[b]], ssem[b]).wait()

    start_idx(0, 0)
    start_idx(1, 1)
    wait_idx(0, 0)
    start_gather(0, 0)
    for i in range(2):
        start_idx(i + 2, (i + 2) % _NBUF)
        wait_idx(i + 1, (i + 1) % _NBUF)
        start_gather(i + 1, (i + 1) % _NBUF)
        wait_gather(i, i % _NBUF)
        start_scatter(i, i % _NBUF)

    n_main = NCHUNK - 5
    assert n_main % _NBUF == 0

    def group(g, carry):
        for k in range(_NBUF):
            i = 2 + g * _NBUF + k
            b = (2 + k) % _NBUF
            b1 = (3 + k) % _NBUF
            b2 = k
            wait_scatter(i - 2, b2)
            start_idx(i + 2, b2)
            wait_idx(i + 1, b1)
            start_gather(i + 1, b1)
            wait_gather(i, b)
            start_scatter(i, b)
        return carry
    lax.fori_loop(0, n_main // _NBUF, group, 0)

    for i in range(NCHUNK - 3, NCHUNK):
        b = i % _NBUF
        if i + 2 < NCHUNK:
            wait_scatter(i - 2, (i + 2) % _NBUF)
            start_idx(i + 2, (i + 2) % _NBUF)
        if i + 1 < NCHUNK:
            wait_idx(i + 1, (i + 1) % _NBUF)
            start_gather(i + 1, (i + 1) % _NBUF)
        wait_gather(i, b)
        start_scatter(i, b)
    for i in range(NCHUNK - 4, NCHUNK):
        wait_scatter(i, i % _NBUF)

    plsc.subcore_barrier()
    pltpu.sync_copy(acc_sh.at[pl.ds(s * ROWS_PT, ROWS_PT)],
                    out_hbm.at[c, pl.ds(s * ROWS_PT, ROWS_PT)])


_BLK = 1000


def _dinv_body(hist_ref, o_ref):
    deg = 1.0 + jnp.sum(hist_ref[...], axis=0, keepdims=True)
    o_ref[...] = lax.rsqrt(deg)


def _dinv_from_hists(hists):
    return pl.pallas_call(
        _dinv_body,
        in_specs=[pl.BlockSpec((NW, NPAD), lambda: (0, 0))],
        out_specs=pl.BlockSpec((1, NPAD), lambda: (0, 0)),
        out_shape=jax.ShapeDtypeStruct((1, NPAD), jnp.float32),
    )(hists)


def _mm_scale_body(x_ref, w_ref, dinv_ref, o_ref):
    o_ref[...] = jnp.dot(x_ref[...], w_ref[...],
                         preferred_element_type=jnp.float32) * dinv_ref[...]


def _mm_scale(x, w, dinv2):
    return pl.pallas_call(
        _mm_scale_body,
        grid=(N // _BLK,),
        in_specs=[
            pl.BlockSpec((_BLK, H), lambda i: (i, 0)),
            pl.BlockSpec((H, H), lambda i: (0, 0)),
            pl.BlockSpec((_BLK, 1), lambda i: (i, 0)),
        ],
        out_specs=pl.BlockSpec((_BLK, H), lambda i: (i, 0)),
        out_shape=jax.ShapeDtypeStruct((N, H), jnp.float32),
    )(x, w, dinv2)


def _mid_body(acc_ref, hh_ref, dinv_ref, w_ref, b_ref, g_ref, be_ref, o_ref):
    dinv = dinv_ref[...]
    a = (acc_ref[0] + acc_ref[1] + hh_ref[...]) * dinv + b_ref[...]
    z = jnp.maximum(a * _BN_C * g_ref[...] + be_ref[...], 0.0)
    o_ref[...] = jnp.dot(z, w_ref[...],
                         preferred_element_type=jnp.float32) * dinv


def _mid_layer(accs, hh, dinv2, w_next, b, g, be):
    return pl.pallas_call(
        _mid_body,
        grid=(N // _BLK,),
        in_specs=[
            pl.BlockSpec((NC, _BLK, H), lambda i: (0, i, 0)),
            pl.BlockSpec((_BLK, H), lambda i: (i, 0)),
            pl.BlockSpec((_BLK, 1), lambda i: (i, 0)),
            pl.BlockSpec((H, H), lambda i: (0, 0)),
            pl.BlockSpec((H,), lambda i: (0,)),
            pl.BlockSpec((H,), lambda i: (0,)),
            pl.BlockSpec((H,), lambda i: (0,)),
        ],
        out_specs=pl.BlockSpec((_BLK, H), lambda i: (i, 0)),
        out_shape=jax.ShapeDtypeStruct((N, H), jnp.float32),
    )(accs, hh, dinv2, w_next, b, g, be)


def _head_body(acc_ref, hh_ref, dinv_ref, b_ref, g_ref, be_ref,
               brow_ref, bcol_ref, wm1_ref, bm1_ref, wm2_ref, bm2_ref,
               wm3_ref, bm3_ref, o_ref, mx_ref):
    a = (acc_ref[0] + acc_ref[1] + hh_ref[...]) * dinv_ref[...] + b_ref[...]
    h = jnp.maximum(a * _BN_C * g_ref[...] + be_ref[...], 0.0)
    brow = brow_ref[...]
    bcol = bcol_ref[...]
    gids = lax.broadcasted_iota(jnp.int32, (G, N), 0)
    onehot = (brow == gids).astype(jnp.float32)
    sums = jnp.dot(onehot, h, preferred_element_type=jnp.float32,
                   precision=lax.Precision.HIGHEST)
    cnt = jnp.sum(onehot, axis=1, keepdims=True)
    mean = sums / jnp.maximum(cnt, 1.0)

    def body(g, carry):
        m = jnp.where(bcol == g, h, -jnp.inf)
        mx_ref[pl.ds(g, 1), :] = jnp.max(m, axis=0, keepdims=True)
        return carry
    lax.fori_loop(0, G, body, 0)

    xg = jnp.concatenate([mean, mx_ref[...]], axis=1)
    o = jnp.maximum(jnp.dot(xg, wm1_ref[...],
                            preferred_element_type=jnp.float32)
                    + bm1_ref[...], 0.0)
    o = jnp.maximum(jnp.dot(o, wm2_ref[...],
                            preferred_element_type=jnp.float32)
                    + bm2_ref[...], 0.0)
    o = jnp.dot(o, wm3_ref[...], preferred_element_type=jnp.float32) \
        + bm3_ref[...]
    o_ref[...] = o[:, 0]


def _head(accs, hh, dinv2, b3, g3, be3, batch, Wm1, bm1, Wm2, bm2, Wm3, bm3):
    return pl.pallas_call(
        _head_body,
        grid=(1,),
        in_specs=[
            pl.BlockSpec((NC, N, H), lambda i: (0, 0, 0)),
            pl.BlockSpec((N, H), lambda i: (0, 0)),
            pl.BlockSpec((N, 1), lambda i: (0, 0)),
            pl.BlockSpec((H,), lambda i: (0,)),
            pl.BlockSpec((H,), lambda i: (0,)),
            pl.BlockSpec((H,), lambda i: (0,)),
            pl.BlockSpec((1, N), lambda i: (0, 0)),
            pl.BlockSpec((N, 1), lambda i: (0, 0)),
            pl.BlockSpec((2 * H, 256), lambda i: (0, 0)),
            pl.BlockSpec((256,), lambda i: (0,)),
            pl.BlockSpec((256, 64), lambda i: (0, 0)),
            pl.BlockSpec((64,), lambda i: (0,)),
            pl.BlockSpec((64, 1), lambda i: (0, 0)),
            pl.BlockSpec((1,), lambda i: (0,)),
        ],
        out_specs=pl.BlockSpec((G,), lambda i: (0,)),
        out_shape=jax.ShapeDtypeStruct((G,), jnp.float32),
        scratch_shapes=[pltpu.VMEM((G, H), jnp.float32)],
    )(accs, hh, dinv2, b3, g3, be3, batch[None, :], batch[:, None],
      Wm1, bm1, Wm2, bm2, Wm3, bm3)


def kernel(x, edge_index, batch, W1, b1, g1, be1, W2, b2, g2, be2,
           W3, b3, g3, be3, Wm1, bm1, Wm2, bm2, Wm3, bm3):
    src = edge_index[0]
    dst = edge_index[1]
    dst3d = dst.reshape(NW, DNCH, DCH)

    hists = _deg_kernel(dst3d)
    dinv_row = _dinv_from_hists(hists)
    dinv2 = dinv_row[0, :N][:, None]

    hh = _mm_scale(x, W1, dinv2)
    accs1 = _agg_kernel(hh, src, dst)
    hh2 = _mid_layer(accs1, hh, dinv2, W2, b1, g1, be1)
    accs2 = _agg_kernel(hh2, src, dst)
    hh3 = _mid_layer(accs2, hh2, dinv2, W3, b2, g2, be2)
    accs3 = _agg_kernel(hh3, src, dst)
    return _head(accs3, hh3, dinv2, b3, g3, be3, batch,
                 Wm1, bm1, Wm2, bm2, Wm3, bm3)
